# Initial kernel scaffold; baseline (speedup 1.0000x reference)
import jax
import jax.numpy as jnp
from jax.experimental import pallas as pl


def kernel(table, inputs):
    raise NotImplementedError("write your pallas kernel here")


# speedup vs baseline: 1.1016x; 1.1016x over previous
import functools

import jax
import jax.numpy as jnp
from jax import lax
from jax.experimental import pallas as pl
from jax.experimental.pallas import tpu as pltpu
from jax.experimental.pallas import tpu_sc as plsc

_NC = 2
_NS = 16
_NW = _NC * _NS


@functools.lru_cache(maxsize=None)
def _make_gather(V, D, N, chunk):
    n_per_w = N // _NW
    n_chunks = n_per_w // chunk
    mesh = plsc.VectorSubcoreMesh(core_axis_name="c", subcore_axis_name="s")

    @functools.partial(
        pl.kernel,
        mesh=mesh,
        out_type=jax.ShapeDtypeStruct((N, D), jnp.float32),
        scratch_types=[
            pltpu.VMEM((chunk,), jnp.int32),
            pltpu.VMEM((chunk, D), jnp.float32),
            pltpu.SemaphoreType.DMA,
        ],
        compiler_params=pltpu.CompilerParams(use_tc_tiling_on_sc=False),
    )
    def k(table_hbm, idx_hbm, out_hbm, idx_v, rows_v, sem):
        wid = lax.axis_index("s") * _NC + lax.axis_index("c")
        base = wid * n_per_w

        def body(i, carry):
            off = base + i * chunk
            pltpu.sync_copy(idx_hbm.at[pl.ds(off, chunk)], idx_v)
            pltpu.async_copy(table_hbm.at[idx_v], rows_v, sem).wait()
            pltpu.sync_copy(rows_v, out_hbm.at[pl.ds(off, chunk)])
            return carry

        lax.fori_loop(0, n_chunks, body, 0)

    return k


def kernel(table, inputs):
    B, H = inputs.shape
    V, D = table.shape
    N = B * H
    idx = inputs.reshape(N).astype(jnp.int32)
    out = _make_gather(V, D, N, 1600)(table, idx)
    return out.reshape(B, H, D)


# speedup vs baseline: 1.1124x; 1.0098x over previous
import functools

import jax
import jax.numpy as jnp
from jax import lax
from jax.experimental import pallas as pl
from jax.experimental.pallas import tpu as pltpu
from jax.experimental.pallas import tpu_sc as plsc

_NC = 2
_NS = 16
_NW = _NC * _NS


@functools.lru_cache(maxsize=None)
def _make_gather(V, D, N, chunk):
    n_per_w = N // _NW
    n_chunks = n_per_w // chunk
    mesh = plsc.VectorSubcoreMesh(core_axis_name="c", subcore_axis_name="s")

    @functools.partial(
        pl.kernel,
        mesh=mesh,
        out_type=jax.ShapeDtypeStruct((N, D), jnp.float32),
        scratch_types=[
            pltpu.VMEM((n_per_w,), jnp.int32),
            pltpu.VMEM((chunk, D), jnp.float32),
            pltpu.VMEM((chunk, D), jnp.float32),
            pltpu.SemaphoreType.DMA,
            pltpu.SemaphoreType.DMA,
        ],
        compiler_params=pltpu.CompilerParams(use_tc_tiling_on_sc=False),
    )
    def k(table_hbm, idx_hbm, out_hbm, idx_v, rows0, rows1, sem0, sem1):
        wid = lax.axis_index("s") * _NC + lax.axis_index("c")
        base = wid * n_per_w
        pltpu.sync_copy(idx_hbm.at[pl.ds(base, n_per_w)], idx_v)

        bufs = (rows0, rows1)
        sems = (sem0, sem1)
        handles = [None, None]
        handles[0] = pltpu.async_copy(
            table_hbm.at[idx_v.at[pl.ds(0, chunk)]], bufs[0], sems[0])
        for i in range(n_chunks):
            b = i % 2
            nb = (i + 1) % 2
            if i + 1 < n_chunks:
                handles[nb] = pltpu.async_copy(
                    table_hbm.at[idx_v.at[pl.ds((i + 1) * chunk, chunk)]],
                    bufs[nb], sems[nb])
            handles[b].wait()
            pltpu.sync_copy(bufs[b], out_hbm.at[pl.ds(base + i * chunk, chunk)])

    return k


def kernel(table, inputs):
    B, H = inputs.shape
    V, D = table.shape
    N = B * H
    idx = inputs.reshape(N).astype(jnp.int32)
    out = _make_gather(V, D, N, 1600)(table, idx)
    return out.reshape(B, H, D)


# speedup vs baseline: 1.8072x; 1.6245x over previous
import functools

import jax
import jax.numpy as jnp
from jax import lax
from jax.experimental import pallas as pl
from jax.experimental.pallas import tpu as pltpu
from jax.experimental.pallas import tpu_sc as plsc

_NC = 2
_NS = 16
_NW = _NC * _NS


@functools.lru_cache(maxsize=None)
def _make_gather(V, D, B, H, bchunk):
    b_per_w = B // _NW
    n_chunks = b_per_w // bchunk
    chunk = bchunk * H
    mesh = plsc.VectorSubcoreMesh(core_axis_name="c", subcore_axis_name="s")

    @functools.partial(
        pl.kernel,
        mesh=mesh,
        out_type=jax.ShapeDtypeStruct((B, H, D), jnp.float32),
        scratch_types=[
            pltpu.VMEM((b_per_w * H,), jnp.int32),
            pltpu.VMEM((chunk, D), jnp.float32),
            pltpu.VMEM((chunk, D), jnp.float32),
            pltpu.SemaphoreType.DMA,
            pltpu.SemaphoreType.DMA,
            pltpu.SemaphoreType.DMA,
            pltpu.SemaphoreType.DMA,
        ],
        compiler_params=pltpu.CompilerParams(use_tc_tiling_on_sc=False),
    )
    def k(table_hbm, idx_hbm, out_hbm, idx_v, rows0, rows1,
          gsem0, gsem1, wsem0, wsem1):
        wid = lax.axis_index("s") * _NC + lax.axis_index("c")
        b_base = wid * b_per_w
        pltpu.sync_copy(idx_hbm.at[pl.ds(b_base * H, b_per_w * H)], idx_v)

        bufs = (rows0, rows1)
        gsems = (gsem0, gsem1)
        wsems = (wsem0, wsem1)

        def gather(i, b):
            return pltpu.async_copy(
                table_hbm.at[idx_v.at[pl.ds(i * chunk, chunk)]],
                bufs[b], gsems[b])

        def drain_writes(b):
            pltpu.make_async_copy(
                table_hbm.at[pl.ds(0, chunk)], bufs[b], wsems[b]).wait()

        gh = [None, None]
        gh[0] = gather(0, 0)
        for i in range(n_chunks):
            b = i % 2
            nb = (i + 1) % 2
            if i + 1 < n_chunks:
                if i >= 1:
                    drain_writes(nb)
                gh[nb] = gather(i + 1, nb)
            gh[b].wait()
            for bl in range(bchunk):
                pltpu.async_copy(
                    bufs[b].at[pl.ds(bl * H, H)],
                    out_hbm.at[b_base + i * bchunk + bl],
                    wsems[b])
        drain_writes(0 if n_chunks % 2 == 1 else 1)
        drain_writes(1 if n_chunks % 2 == 1 else 0)

    return k


def kernel(table, inputs):
    B, H = inputs.shape
    V, D = table.shape
    idx = inputs.reshape(B * H).astype(jnp.int32)
    return _make_gather(V, D, B, H, 32)(table, idx)
